# Initial kernel scaffold; baseline (speedup 1.0000x reference)
import jax
import jax.numpy as jnp
from jax.experimental import pallas as pl


def kernel(days_index, events, W, b):
    raise NotImplementedError("write your pallas kernel here")


# speedup vs baseline: 1.1651x; 1.1651x over previous
import functools

import jax
import jax.numpy as jnp
from jax import lax
from jax.experimental import pallas as pl
from jax.experimental.pallas import tpu as pltpu
from jax.experimental.pallas import tpu_sc as plsc

_NUM_CORES = 2
_NUM_SUBCORES = 16
_NUM_WORKERS = _NUM_CORES * _NUM_SUBCORES

_NUM_EVENTS = 1969
_BATCH = 16384
_D_PAD = 16
_B_PER_W = _BATCH // _NUM_WORKERS


def _project_body(ev_ref, w_ref, b_ref, out_ref):
    out_ref[...] = (
        jnp.dot(ev_ref[...], w_ref[...], preferred_element_type=jnp.float32)
        + b_ref[...]
    )


def _project(events, w_pad, b_pad):
    return pl.pallas_call(
        _project_body,
        out_shape=jax.ShapeDtypeStruct((_NUM_EVENTS, _D_PAD), jnp.float32),
    )(events, w_pad, b_pad)


_sc_mesh = plsc.VectorSubcoreMesh(
    core_axis_name="c",
    subcore_axis_name="s",
    num_cores=_NUM_CORES,
    num_subcores=_NUM_SUBCORES,
)


@functools.partial(
    pl.kernel,
    out_type=jax.ShapeDtypeStruct((_BATCH, _D_PAD), jnp.float32),
    mesh=_sc_mesh,
    scratch_types=[
        pltpu.VMEM((_B_PER_W,), jnp.int32),
        pltpu.VMEM((_B_PER_W, _D_PAD), jnp.float32),
        pltpu.SemaphoreType.DMA,
    ],
    compiler_params=pltpu.CompilerParams(use_tc_tiling_on_sc=False),
)
def _gather_rows(table_hbm, idx_hbm, out_hbm, idx_v, rows_v, sem):
    wid = lax.axis_index("s") * _NUM_CORES + lax.axis_index("c")
    base = wid * _B_PER_W
    pltpu.sync_copy(idx_hbm.at[pl.ds(base, _B_PER_W)], idx_v)
    pltpu.async_copy(table_hbm.at[idx_v], rows_v, sem).wait()
    pltpu.sync_copy(rows_v, out_hbm.at[pl.ds(base, _B_PER_W)])


def kernel(days_index, events, W, b):
    w_pad = jnp.zeros((31, _D_PAD), jnp.float32).at[:, :5].set(W)
    b_pad = jnp.zeros((1, _D_PAD), jnp.float32).at[0, :5].set(b)
    table = _project(events, w_pad, b_pad)
    gathered = _gather_rows(table, days_index)
    return gathered[:, :5]
